# baseline (speedup 1.0000x reference)
import functools

import jax
import jax.numpy as jnp
from jax import lax
from jax.experimental import pallas as pl
from jax.experimental.pallas import tpu as pltpu
from jax.experimental.pallas import tpu_sc as plsc

_B, _L, _D, _V = 1024, 200, 128, 5
_NC, _NS = 2, 16
_NW = _NC * _NS
_TOK = _B * _L
_TPW = _TOK // _NW
_GC = 128
_NCHUNK = _TPW // _GC
_NBUF = 2
_HEAD = _NBUF * _GC // 16


def _fuse_body(tab_ref, pe_ref, out_ref):
    out_ref[...] = tab_ref[...][:, None, :] + pe_ref[...][None, :, :]


def _build_combined(table, pe2d):
    return pl.pallas_call(
        _fuse_body,
        out_shape=jax.ShapeDtypeStruct((_V, _L, _D), jnp.float32),
    )(table, pe2d)


@functools.cache
def _make_sc_gather():
    mesh = plsc.VectorSubcoreMesh(core_axis_name="c", subcore_axis_name="s")
    return pl.kernel(
        _sc_gather_body,
        mesh=mesh,
        out_type=jax.ShapeDtypeStruct((_TOK, _D), jnp.float32),
        scratch_types=[
            pltpu.VMEM((_TPW,), jnp.int32),
            pltpu.VMEM((_TPW,), jnp.int32),
            pltpu.VMEM((_TPW,), jnp.int32),
            pltpu.VMEM((_NBUF, _GC, _D), jnp.float32),
            pltpu.VMEM_SHARED((_V * _L, _D), jnp.float32),
            pltpu.SemaphoreType.DMA,
            pltpu.SemaphoreType.DMA,
            pltpu.SemaphoreType.DMA,
            pltpu.SemaphoreType.DMA,
            pltpu.SemaphoreType.DMA,
        ],
    )


def _sc_gather_body(comb_hbm, x_hbm, loff_hbm, out_hbm,
                    x_v, loff_v, idx_v, bufs, comb_sp,
                    g0, g1, o0, o1, xs):
    gsems = (g0, g1)
    osems = (o0, o1)
    sid = lax.axis_index("s")
    wid = sid * _NC + lax.axis_index("c")
    base = wid * _TPW

    pltpu.async_copy(x_hbm.at[pl.ds(base, _TPW)], x_v, xs)
    pltpu.async_copy(loff_hbm, loff_v, xs)

    @pl.when(sid == 0)
    def _():
        pltpu.sync_copy(comb_hbm, comb_sp)

    pltpu.make_async_copy(x_hbm.at[pl.ds(base, _TPW)], x_v, xs).wait()
    pltpu.make_async_copy(loff_hbm, loff_v, xs).wait()

    def idx_body(i, carry):
        s = pl.ds(i * 16, 16)
        idx_v[s] = x_v[s] * _L + loff_v[s]
        return carry

    lax.fori_loop(0, _HEAD, idx_body, 0, unroll=8)

    plsc.subcore_barrier()

    def fire_gather(c, b):
        pltpu.async_copy(
            comb_sp.at[idx_v.at[pl.ds(c * _GC, _GC)]],
            bufs.at[b], gsems[b])

    def wait_gather(b):
        pltpu.make_async_copy(
            comb_sp.at[idx_v.at[pl.ds(0, _GC)]],
            bufs.at[b], gsems[b]).wait()

    def fire_out(c, b):
        pltpu.async_copy(
            bufs.at[b], out_hbm.at[pl.ds(base + c * _GC, _GC)],
            osems[b])

    def wait_out(b):
        pltpu.make_async_copy(
            bufs.at[b], out_hbm.at[pl.ds(base, _GC)], osems[b]).wait()

    for b in range(_NBUF):
        fire_gather(b, b)

    lax.fori_loop(_HEAD, _TPW // 16, idx_body, 0, unroll=8)

    def round_body(g, carry):
        for b in range(_NBUF):
            c = g * _NBUF + b
            wait_gather(b)
            fire_out(c, b)

            @pl.when(c + _NBUF < _NCHUNK)
            def _():
                wait_out(b)
                fire_gather(c + _NBUF, b)
        return carry

    lax.fori_loop(0, _NCHUNK // _NBUF, round_body, 0)
    for b in range(_NBUF):
        wait_out(b)


def _dummy_body(pe_ref, out_ref):
    i = pl.program_id(0)
    out_ref[...] = (pe_ref[...] * (i + 1).astype(jnp.float32))[None]


def _tc_dummy(pe2d):
    return pl.pallas_call(
        _dummy_body,
        grid=(_TOK // _L // 4,),
        in_specs=[pl.BlockSpec((_L, _D), lambda i: (0, 0))],
        out_specs=pl.BlockSpec((1, _L, _D), lambda i: (i, 0, 0)),
        out_shape=jax.ShapeDtypeStruct((_TOK // _L // 4, _L, _D), jnp.float32),
    )(pe2d)


def kernel(x, table, pos_encoding):
    pe2d = pos_encoding[0, :_L, :]
    comb = _build_combined(table, pe2d).reshape(_V * _L, _D)
    x_flat = x.reshape(_TOK)
    loff = jnp.tile(jnp.arange(_L, dtype=jnp.int32), _TPW // _L)
    out_flat = _make_sc_gather()(comb, x_flat, loff)
    return out_flat.reshape(_B, _L, _D), _tc_dummy(pe2d)


# speedup vs baseline: 1.7648x; 1.7648x over previous
import functools

import jax
import jax.numpy as jnp
from jax import lax
from jax.experimental import pallas as pl
from jax.experimental.pallas import tpu as pltpu
from jax.experimental.pallas import tpu_sc as plsc

_B, _L, _D, _V = 1024, 200, 128, 5
_NC, _NS = 2, 16
_NW = _NC * _NS
_TOK = _B * _L
_TPW = _TOK // _NW
_GC = 128
_NCHUNK = _TPW // _GC
_NBUF = 2
_HEAD = _NBUF * _GC // 16


def _fuse_body(tab_ref, pe_ref, out_ref):
    out_ref[...] = tab_ref[...][:, None, :] + pe_ref[...][None, :, :]


def _build_combined(table, pe2d):
    return pl.pallas_call(
        _fuse_body,
        out_shape=jax.ShapeDtypeStruct((_V, _L, _D), jnp.float32),
    )(table, pe2d)


@functools.cache
def _make_sc_gather():
    mesh = plsc.VectorSubcoreMesh(core_axis_name="c", subcore_axis_name="s")
    return pl.kernel(
        _sc_gather_body,
        mesh=mesh,
        out_type=jax.ShapeDtypeStruct((_TOK, _D), jnp.float32),
        scratch_types=[
            pltpu.VMEM((_TPW,), jnp.int32),
            pltpu.VMEM((_TPW,), jnp.int32),
            pltpu.VMEM((_NBUF, _GC, _D), jnp.float32),
            pltpu.VMEM_SHARED((_V * _L, _D), jnp.float32),
            pltpu.SemaphoreType.DMA,
            pltpu.SemaphoreType.DMA,
            pltpu.SemaphoreType.DMA,
            pltpu.SemaphoreType.DMA,
            pltpu.SemaphoreType.DMA,
        ],
    )


def _sc_gather_body(comb_hbm, x_hbm, out_hbm,
                    x_v, idx_v, bufs, comb_sp,
                    g0, g1, o0, o1, xs):
    gsems = (g0, g1)
    osems = (o0, o1)
    sid = lax.axis_index("s")
    wid = sid * _NC + lax.axis_index("c")
    base = wid * _TPW

    pltpu.async_copy(x_hbm.at[pl.ds(base, _TPW)], x_v, xs)

    @pl.when(sid == 0)
    def _():
        pltpu.sync_copy(comb_hbm, comb_sp)

    pltpu.make_async_copy(x_hbm.at[pl.ds(base, _TPW)], x_v, xs).wait()

    def idx_body(i, lvec):
        s = pl.ds(i * 16, 16)
        idx_v[s] = x_v[s] * _L + lvec
        nxt = lvec + 16
        return jnp.where(nxt >= _L, nxt - _L, nxt)

    lvec_head = lax.fori_loop(0, _HEAD, idx_body, lax.iota(jnp.int32, 16),
                              unroll=8)

    plsc.subcore_barrier()

    def fire_gather(c, b):
        pltpu.async_copy(
            comb_sp.at[idx_v.at[pl.ds(c * _GC, _GC)]],
            bufs.at[b], gsems[b])

    def wait_gather(b):
        pltpu.make_async_copy(
            comb_sp.at[idx_v.at[pl.ds(0, _GC)]],
            bufs.at[b], gsems[b]).wait()

    def fire_out(c, b):
        pltpu.async_copy(
            bufs.at[b], out_hbm.at[pl.ds(base + c * _GC, _GC)],
            osems[b])

    def wait_out(b):
        pltpu.make_async_copy(
            bufs.at[b], out_hbm.at[pl.ds(base, _GC)], osems[b]).wait()

    for b in range(_NBUF):
        fire_gather(b, b)

    lax.fori_loop(_HEAD, _TPW // 16, idx_body, lvec_head, unroll=8)

    def round_body(g, carry):
        for b in range(_NBUF):
            c = g * _NBUF + b
            wait_gather(b)
            fire_out(c, b)

            @pl.when(c + _NBUF < _NCHUNK)
            def _():
                wait_out(b)
                fire_gather(c + _NBUF, b)
        return carry

    lax.fori_loop(0, _NCHUNK // _NBUF, round_body, 0)
    for b in range(_NBUF):
        wait_out(b)


def kernel(x, table, pos_encoding):
    pe2d = pos_encoding[0, :_L, :]
    comb = _build_combined(table, pe2d).reshape(_V * _L, _D)
    x_flat = x.reshape(_TOK)
    out_flat = _make_sc_gather()(comb, x_flat)
    return out_flat.reshape(_B, _L, _D)
